# baseline (speedup 1.0000x reference)
import functools

import numpy as np
import jax
import jax.numpy as jnp
from jax import lax
from jax.experimental import pallas as pl
from jax.experimental.pallas import tpu as pltpu

_NUM_CLASSES = 80
_IMG = 512.0
_STRIDES = (8, 16, 32)
_SIZES = ((64, 64), (32, 32), (16, 16))
_LAMBDA_BOX = 5.0
_LAMBDA_OBJ = 1.0
_LAMBDA_CLS = 0.5
_ASSIGN_CLS_W = 0.5
_CENTER_RADIUS = 2.0
_TOPK = 20
_CLS_SMOOTH = 0.05
_AREA_MIN = 4.0 / 1.25
_AREA_MAX = 256.0 * 1.25
_SIZE_PRIOR_W = 0.2
_AR_PRIOR_W = 0.1
_IOU_COST_W = 3.0
_CENTER_COST_W = 0.5
_EPS = 1e-7
_PI2 = float(np.pi) ** 2
_A = sum(h * w for h, w in _SIZES)
_G = 30
_BIGI = np.int32(2 ** 30)


def _anchor_consts():
    apx, apy, st = [], [], []
    for (h, w), s in zip(_SIZES, _STRIDES):
        ys, xs = np.meshgrid(np.arange(h, dtype=np.float32),
                             np.arange(w, dtype=np.float32), indexing="ij")
        apx.append((xs.reshape(-1) + 0.5) * s)
        apy.append((ys.reshape(-1) + 0.5) * s)
        st.append(np.full((h * w,), float(s), dtype=np.float32))
    return (np.concatenate(apx)[None, :], np.concatenate(apy)[None, :],
            np.concatenate(st)[None, :])


_APX_NP, _APY_NP, _ST_NP = _anchor_consts()


def _sigmoid(x):
    return 1.0 / (1.0 + jnp.exp(-x))


def _softplus(x):
    return jnp.maximum(x, 0.0) + jnp.log1p(jnp.exp(-jnp.abs(x)))


def _bce(logits, targets):
    return (jnp.maximum(logits, 0.0) - logits * targets
            + jnp.log1p(jnp.exp(-jnp.abs(logits))))


def _atan_pos(x):
    big = x > 2.414213562373095
    mid = x > 0.4142135623730950
    xr = jnp.where(big, -1.0 / x, jnp.where(mid, (x - 1.0) / (x + 1.0), x))
    base = jnp.where(big, 0.5 * np.pi, jnp.where(mid, 0.25 * np.pi, 0.0))
    z = xr * xr
    p = 8.05374449538e-2
    p = p * z - 1.38776856032e-1
    p = p * z + 1.99777106478e-1
    p = p * z - 3.33329491539e-1
    return base + xr + xr * z * p


def _img_kernel(p3_ref, p4_ref, p5_ref, gtb_ref, gtt_ref, lab_ref,
                apx_ref, apy_ref, st_ref, out_ref, uni_ref):
    pred = jnp.concatenate([p3_ref[0], p4_ref[0], p5_ref[0]], axis=1)
    apx = apx_ref[...]
    apy = apy_ref[...]
    st = st_ref[...]

    obj = pred[4:5]
    cls = pred[5:5 + _NUM_CLASSES]

    cx = (2.0 * _sigmoid(pred[0:1]) - 1.0) * st + apx
    cy = (2.0 * _sigmoid(pred[1:2]) - 1.0) * st + apy
    pw = _softplus(pred[2:3]) * st
    ph = _softplus(pred[3:4]) * st
    px1 = cx - 0.5 * pw
    px2 = cx + 0.5 * pw
    py1 = cy - 0.5 * ph
    py2 = cy + 0.5 * ph

    gtb = gtb_ref[0]
    gcx = gtb[:, 0:1] * _IMG
    gcy = gtb[:, 1:2] * _IMG
    gww = gtb[:, 2:3] * _IMG
    ghh = gtb[:, 3:4] * _IMG
    gx1 = gcx - 0.5 * gww
    gx2 = gcx + 0.5 * gww
    gy1 = gcy - 0.5 * ghh
    gy2 = gcy + 0.5 * ghh
    gw = jnp.clip(gx2 - gx1, _EPS)
    gh = jnp.clip(gy2 - gy1, _EPS)

    inv_st2 = 1.0 / (st * st)
    area_cells = (gw * gh) * inv_st2
    gate = (area_cells >= _AREA_MIN) & (area_cells <= _AREA_MAX)
    dx = jnp.abs(apx - gcx)
    dy = jnp.abs(apy - gcy)
    r = _CENTER_RADIUS * st
    in_center = (dx < r) & (dy < r)
    in_box = (apx > gx1) & (apx < gx2) & (apy > gy1) & (apy < gy2)
    cand = gate & (in_center | in_box)

    a1 = jnp.clip(gx2 - gx1, 0.0) * jnp.clip(gy2 - gy1, 0.0)
    a2 = jnp.clip(px2 - px1, 0.0) * jnp.clip(py2 - py1, 0.0)
    ix1 = jnp.maximum(gx1, px1)
    iy1 = jnp.maximum(gy1, py1)
    ix2 = jnp.minimum(gx2, px2)
    iy2 = jnp.minimum(gy2, py2)
    inter = jnp.clip(ix2 - ix1, 0.0) * jnp.clip(iy2 - iy1, 0.0)
    iou = jnp.clip(inter / (a1 + a2 - inter + _EPS), 0.0, 1.0)

    pobj = _sigmoid(obj)
    t_exp = jnp.exp(-cls)
    num = (1.0 + t_exp) * (1.0 + _EPS) - pobj
    den = 1.0 + t_exp
    n1 = num[0:40] * num[40:80]
    n2 = n1[0:20] * n1[20:40]
    n3 = n2[0:10] * n2[10:20]
    d1 = den[0:40] * den[40:80]
    d2 = d1[0:20] * d1[20:40]
    d3 = d2[0:10] * d2[10:20]
    s_all = (jnp.sum(jnp.log(n3), axis=0, keepdims=True)
             - jnp.sum(jnp.log(d3), axis=0, keepdims=True))

    labs = lab_ref[0]
    c_iota = lax.broadcasted_iota(jnp.int32, (_NUM_CLASSES, _G), 0)
    oh_t = (labs == c_iota).astype(jnp.float32)
    dims = (((0,), (0,)), ((), ()))
    oh_b = oh_t.astype(jnp.bfloat16)
    c_hi = cls.astype(jnp.bfloat16)
    c_lo = (cls - c_hi.astype(jnp.float32)).astype(jnp.bfloat16)
    clsg = (lax.dot_general(oh_b, c_hi, dims,
                            preferred_element_type=jnp.float32)
            + lax.dot_general(oh_b, c_lo, dims,
                              preferred_element_type=jnp.float32))
    pg = _sigmoid(clsg) * pobj
    cls_sel = jnp.log(pg + _EPS) - jnp.log(1.0 - pg + _EPS)
    cls_cost = -(cls_sel + s_all)

    lpw = jnp.log(jnp.clip(px2 - px1, _EPS))
    lph = jnp.log(jnp.clip(py2 - py1, _EPS))
    lgw = jnp.log(gw)
    lgh = jnp.log(gh)
    size_cost = jnp.abs(lpw - lgw) + jnp.abs(lph - lgh)
    ar_cost = jnp.abs((lpw - lph) - (lgw - lgh))
    center_cost = jnp.sqrt(dx * dx + dy * dy) / st

    cost = (_IOU_COST_W * (1.0 - iou) + _ASSIGN_CLS_W * cls_cost
            + _SIZE_PRIOR_W * size_cost + _AR_PRIOR_W * ar_cost
            + _CENTER_COST_W * center_cost
            + jnp.where(cand, 0.0, 1e5))

    inf2 = jnp.full((2, _A), jnp.inf, jnp.float32)
    uni_ref[...] = jnp.concatenate(
        [-jnp.where(cand, iou, 0.0), inf2,
         jnp.where(cand, cost, jnp.inf), inf2], axis=0)
    cid20 = lax.broadcasted_iota(jnp.int32, (2 * _G + 4, _TOPK), 1)

    def ext_body(rr, carry):
        prev, mins = carry
        w = uni_ref[...]
        mn = jnp.min(jnp.where(w > prev, w, jnp.inf), axis=1,
                     keepdims=True)
        return mn, jnp.where(cid20 == rr, mn, mins)

    _, mins = lax.fori_loop(
        0, _TOPK, ext_body,
        (jnp.full((2 * _G + 4, 1), -jnp.inf, jnp.float32),
         jnp.full((2 * _G + 4, _TOPK), jnp.inf, jnp.float32)))
    tsum = jnp.sum(jnp.maximum(-mins[0:_G], 0.0), axis=1, keepdims=True)
    dyn_k = jnp.clip(tsum.astype(jnp.int32), 1, _TOPK)
    j20 = lax.broadcasted_iota(jnp.int32, (_G, _TOPK), 1)
    tau = jnp.sum(jnp.where(j20 == dyn_k - 1, mins[_G + 2:2 * _G + 2], 0.0),
                  axis=1, keepdims=True)
    matched = cand & (cost <= tau)

    cost_m = jnp.where(matched, cost, jnp.inf)
    mn0 = jnp.min(cost_m, axis=0, keepdims=True)
    gmask = (cost_m == mn0).astype(jnp.float32)
    fg = jnp.max(matched.astype(jnp.float32), axis=0, keepdims=True)

    iou_at = jnp.sum(iou * gmask, axis=0, keepdims=True)
    xlab = jnp.sum(clsg * gmask, axis=0, keepdims=True)
    gtt = gtt_ref[0]
    gdims = (((1,), (0,)), ((), ()))
    g_hi = gtt.astype(jnp.bfloat16)
    g_lo = (gtt - g_hi.astype(jnp.float32)).astype(jnp.bfloat16)
    gm_b = gmask.astype(jnp.bfloat16)
    tcoord = (lax.dot_general(g_hi, gm_b, gdims,
                              preferred_element_type=jnp.float32)
              + lax.dot_general(g_lo, gm_b, gdims,
                                preferred_element_type=jnp.float32))
    tx1 = tcoord[0:1]
    tx2 = tcoord[1:2]
    ty1 = tcoord[2:3]
    ty2 = tcoord[3:4]

    nfg = jnp.maximum(jnp.sum(fg), 1.0)

    pwc = jnp.clip(px2 - px1, _EPS)
    phc = jnp.clip(py2 - py1, _EPS)
    twc = jnp.clip(tx2 - tx1, _EPS)
    thc = jnp.clip(ty2 - ty1, _EPS)
    iw = jnp.clip(jnp.minimum(px2, tx2) - jnp.maximum(px1, tx1), 0.0)
    ih = jnp.clip(jnp.minimum(py2, ty2) - jnp.maximum(py1, ty1), 0.0)
    inter2 = iw * ih
    union2 = pwc * phc + twc * thc - inter2 + _EPS
    iou2 = inter2 / union2
    cd = (((px1 + px2) * 0.5 - (tx1 + tx2) * 0.5) ** 2
          + ((py1 + py2) * 0.5 - (ty1 + ty2) * 0.5) ** 2)
    cw = jnp.maximum(px2, tx2) - jnp.minimum(px1, tx1)
    chh = jnp.maximum(py2, ty2) - jnp.minimum(py1, ty1)
    c2 = cw * cw + chh * chh + _EPS
    v = 4.0 / _PI2 * (_atan_pos(twc / thc) - _atan_pos(pwc / phc)) ** 2
    alpha = v / (v - iou2 + 1.0 + _EPS)
    ciou = iou2 - cd / c2 - alpha * v
    loss_box = jnp.sum(fg * (1.0 - ciou)) / nfg

    obj_t = fg * iou_at
    loss_obj = jnp.sum(_bce(obj, obj_t)) / float(_A)
    off = _CLS_SMOOTH / (_NUM_CLASSES - 1)
    scale = 1.0 - _CLS_SMOOTH - off
    e_abs = jnp.exp(-jnp.abs(cls))
    op1 = 1.0 + e_abs
    w1 = op1[0:40] * op1[40:80]
    w2 = w1[0:20] * w1[20:40]
    w3 = w2[0:10] * w2[10:20]
    sp_sum = (jnp.sum(jnp.maximum(cls, 0.0), axis=0, keepdims=True)
              + jnp.sum(jnp.log(w3), axis=0, keepdims=True))
    sumx = jnp.sum(cls, axis=0, keepdims=True)
    bce_mean = (sp_sum - off * sumx - scale * xlab) / float(_NUM_CLASSES)
    loss_cls = jnp.sum(fg * bce_mean) / nfg

    loss = (_LAMBDA_BOX * loss_box + _LAMBDA_OBJ * loss_obj
            + _LAMBDA_CLS * loss_cls)
    out_ref[...] = loss[None, None, None]


def kernel(p3, p4, p5, gt_boxes, gt_labels):
    b = p3.shape[0]
    p3f = p3.reshape(b, 5 + _NUM_CLASSES, _SIZES[0][0] * _SIZES[0][1])
    p4f = p4.reshape(b, 5 + _NUM_CLASSES, _SIZES[1][0] * _SIZES[1][1])
    p5f = p5.reshape(b, 5 + _NUM_CLASSES, _SIZES[2][0] * _SIZES[2][1])
    labs = gt_labels.astype(jnp.int32).reshape(b, 1, _G)
    cxg = gt_boxes[..., 0] * _IMG
    cyg = gt_boxes[..., 1] * _IMG
    wwg = gt_boxes[..., 2] * _IMG
    hhg = gt_boxes[..., 3] * _IMG
    gtt = jnp.stack([cxg - 0.5 * wwg, cxg + 0.5 * wwg,
                     cyg - 0.5 * hhg, cyg + 0.5 * hhg], axis=1)
    apx = jnp.asarray(_APX_NP)
    apy = jnp.asarray(_APY_NP)
    st = jnp.asarray(_ST_NP)
    ch = 5 + _NUM_CLASSES
    out = pl.pallas_call(
        _img_kernel,
        grid=(b,),
        in_specs=[
            pl.BlockSpec((1, ch, p3f.shape[2]), lambda i: (i, 0, 0)),
            pl.BlockSpec((1, ch, p4f.shape[2]), lambda i: (i, 0, 0)),
            pl.BlockSpec((1, ch, p5f.shape[2]), lambda i: (i, 0, 0)),
            pl.BlockSpec((1, _G, 4), lambda i: (i, 0, 0)),
            pl.BlockSpec((1, 4, _G), lambda i: (i, 0, 0)),
            pl.BlockSpec((1, 1, _G), lambda i: (i, 0, 0)),
            pl.BlockSpec((1, _A), lambda i: (0, 0)),
            pl.BlockSpec((1, _A), lambda i: (0, 0)),
            pl.BlockSpec((1, _A), lambda i: (0, 0)),
        ],
        out_specs=pl.BlockSpec((1, 1, 1), lambda i: (i, 0, 0)),
        out_shape=jax.ShapeDtypeStruct((b, 1, 1), jnp.float32),
        scratch_shapes=[pltpu.VMEM((2 * _G + 4, _A), jnp.float32)],
    )(p3f, p4f, p5f, gt_boxes, gtt, labs, apx, apy, st)
    return jnp.mean(out)


# speedup vs baseline: 1.0043x; 1.0043x over previous
import functools

import numpy as np
import jax
import jax.numpy as jnp
from jax import lax
from jax.experimental import pallas as pl
from jax.experimental.pallas import tpu as pltpu

_NUM_CLASSES = 80
_IMG = 512.0
_STRIDES = (8, 16, 32)
_SIZES = ((64, 64), (32, 32), (16, 16))
_LAMBDA_BOX = 5.0
_LAMBDA_OBJ = 1.0
_LAMBDA_CLS = 0.5
_ASSIGN_CLS_W = 0.5
_CENTER_RADIUS = 2.0
_TOPK = 20
_CLS_SMOOTH = 0.05
_AREA_MIN = 4.0 / 1.25
_AREA_MAX = 256.0 * 1.25
_SIZE_PRIOR_W = 0.2
_AR_PRIOR_W = 0.1
_IOU_COST_W = 3.0
_CENTER_COST_W = 0.5
_EPS = 1e-7
_PI2 = float(np.pi) ** 2
_A = sum(h * w for h, w in _SIZES)
_G = 30
_BIGI = np.int32(2 ** 30)


def _anchor_consts():
    apx, apy, st = [], [], []
    for (h, w), s in zip(_SIZES, _STRIDES):
        ys, xs = np.meshgrid(np.arange(h, dtype=np.float32),
                             np.arange(w, dtype=np.float32), indexing="ij")
        apx.append((xs.reshape(-1) + 0.5) * s)
        apy.append((ys.reshape(-1) + 0.5) * s)
        st.append(np.full((h * w,), float(s), dtype=np.float32))
    return (np.concatenate(apx)[None, :], np.concatenate(apy)[None, :],
            np.concatenate(st)[None, :])


_APX_NP, _APY_NP, _ST_NP = _anchor_consts()


def _sigmoid(x):
    return 1.0 / (1.0 + jnp.exp(-x))


def _softplus(x):
    return jnp.maximum(x, 0.0) + jnp.log1p(jnp.exp(-jnp.abs(x)))


def _bce(logits, targets):
    return (jnp.maximum(logits, 0.0) - logits * targets
            + jnp.log1p(jnp.exp(-jnp.abs(logits))))


def _atan_pos(x):
    big = x > 2.414213562373095
    mid = x > 0.4142135623730950
    xr = jnp.where(big, -1.0 / x, jnp.where(mid, (x - 1.0) / (x + 1.0), x))
    base = jnp.where(big, 0.5 * np.pi, jnp.where(mid, 0.25 * np.pi, 0.0))
    z = xr * xr
    p = 8.05374449538e-2
    p = p * z - 1.38776856032e-1
    p = p * z + 1.99777106478e-1
    p = p * z - 3.33329491539e-1
    return base + xr + xr * z * p


def _img_kernel(p3_ref, p4_ref, p5_ref, gtb_ref, gtt_ref, lab_ref,
                apx_ref, apy_ref, st_ref, out_ref, uni_ref):
    pred = jnp.concatenate([p3_ref[0], p4_ref[0], p5_ref[0]], axis=1)
    apx = apx_ref[...]
    apy = apy_ref[...]
    st = st_ref[...]

    obj = pred[4:5]
    cls = pred[5:5 + _NUM_CLASSES]

    cx = (2.0 * _sigmoid(pred[0:1]) - 1.0) * st + apx
    cy = (2.0 * _sigmoid(pred[1:2]) - 1.0) * st + apy
    pw = _softplus(pred[2:3]) * st
    ph = _softplus(pred[3:4]) * st
    px1 = cx - 0.5 * pw
    px2 = cx + 0.5 * pw
    py1 = cy - 0.5 * ph
    py2 = cy + 0.5 * ph

    gtb = gtb_ref[0]
    gcx = gtb[:, 0:1] * _IMG
    gcy = gtb[:, 1:2] * _IMG
    gww = gtb[:, 2:3] * _IMG
    ghh = gtb[:, 3:4] * _IMG
    gx1 = gcx - 0.5 * gww
    gx2 = gcx + 0.5 * gww
    gy1 = gcy - 0.5 * ghh
    gy2 = gcy + 0.5 * ghh
    gw = jnp.clip(gx2 - gx1, _EPS)
    gh = jnp.clip(gy2 - gy1, _EPS)

    inv_st2 = 1.0 / (st * st)
    area_cells = (gw * gh) * inv_st2
    gate = (area_cells >= _AREA_MIN) & (area_cells <= _AREA_MAX)
    dx = jnp.abs(apx - gcx)
    dy = jnp.abs(apy - gcy)
    r = _CENTER_RADIUS * st
    in_center = (dx < r) & (dy < r)
    in_box = (apx > gx1) & (apx < gx2) & (apy > gy1) & (apy < gy2)
    cand = gate & (in_center | in_box)

    a1 = jnp.clip(gx2 - gx1, 0.0) * jnp.clip(gy2 - gy1, 0.0)
    a2 = jnp.clip(px2 - px1, 0.0) * jnp.clip(py2 - py1, 0.0)
    ix1 = jnp.maximum(gx1, px1)
    iy1 = jnp.maximum(gy1, py1)
    ix2 = jnp.minimum(gx2, px2)
    iy2 = jnp.minimum(gy2, py2)
    inter = jnp.clip(ix2 - ix1, 0.0) * jnp.clip(iy2 - iy1, 0.0)
    iou = jnp.clip(inter / (a1 + a2 - inter + _EPS), 0.0, 1.0)

    pobj = _sigmoid(obj)
    t_exp = jnp.exp(-cls)
    sig_c = 1.0 / (1.0 + t_exp)
    omp = 1.0 - sig_c * pobj + _EPS
    y1 = omp[0:40] * omp[40:80]
    y2 = y1[0:20] * y1[20:40]
    y3 = y2[0:10] * y2[10:20]
    s_all = jnp.sum(jnp.log(y3), axis=0, keepdims=True)

    labs = lab_ref[0]
    c_iota = lax.broadcasted_iota(jnp.int32, (_NUM_CLASSES, _G), 0)
    oh_t = (labs == c_iota).astype(jnp.float32)
    dims = (((0,), (0,)), ((), ()))
    oh_b = oh_t.astype(jnp.bfloat16)
    c_hi = cls.astype(jnp.bfloat16)
    c_lo = (cls - c_hi.astype(jnp.float32)).astype(jnp.bfloat16)
    clsg = (lax.dot_general(oh_b, c_hi, dims,
                            preferred_element_type=jnp.float32)
            + lax.dot_general(oh_b, c_lo, dims,
                              preferred_element_type=jnp.float32))
    pg = _sigmoid(clsg) * pobj
    cls_sel = jnp.log(pg + _EPS) - jnp.log(1.0 - pg + _EPS)
    cls_cost = -(cls_sel + s_all)

    lpw = jnp.log(jnp.clip(px2 - px1, _EPS))
    lph = jnp.log(jnp.clip(py2 - py1, _EPS))
    lgw = jnp.log(gw)
    lgh = jnp.log(gh)
    size_cost = jnp.abs(lpw - lgw) + jnp.abs(lph - lgh)
    ar_cost = jnp.abs((lpw - lph) - (lgw - lgh))
    center_cost = jnp.sqrt(dx * dx + dy * dy) / st

    cost = (_IOU_COST_W * (1.0 - iou) + _ASSIGN_CLS_W * cls_cost
            + _SIZE_PRIOR_W * size_cost + _AR_PRIOR_W * ar_cost
            + _CENTER_COST_W * center_cost
            + jnp.where(cand, 0.0, 1e5))

    inf2 = jnp.full((2, _A), jnp.inf, jnp.float32)
    uni_ref[...] = jnp.concatenate(
        [-jnp.where(cand, iou, 0.0), inf2,
         jnp.where(cand, cost, jnp.inf), inf2], axis=0)
    cid20 = lax.broadcasted_iota(jnp.int32, (2 * _G + 4, _TOPK), 1)

    def ext_body(rr, carry):
        prev, mins = carry
        w = uni_ref[...]
        mn = jnp.min(jnp.where(w > prev, w, jnp.inf), axis=1,
                     keepdims=True)
        return mn, jnp.where(cid20 == rr, mn, mins)

    _, mins = lax.fori_loop(
        0, _TOPK, ext_body,
        (jnp.full((2 * _G + 4, 1), -jnp.inf, jnp.float32),
         jnp.full((2 * _G + 4, _TOPK), jnp.inf, jnp.float32)))
    tsum = jnp.sum(jnp.maximum(-mins[0:_G], 0.0), axis=1, keepdims=True)
    dyn_k = jnp.clip(tsum.astype(jnp.int32), 1, _TOPK)
    j20 = lax.broadcasted_iota(jnp.int32, (_G, _TOPK), 1)
    tau = jnp.sum(jnp.where(j20 == dyn_k - 1, mins[_G + 2:2 * _G + 2], 0.0),
                  axis=1, keepdims=True)
    matched = cand & (cost <= tau)

    cost_m = jnp.where(matched, cost, jnp.inf)
    mn0 = jnp.min(cost_m, axis=0, keepdims=True)
    gmask = (cost_m == mn0).astype(jnp.float32)
    fg = jnp.max(matched.astype(jnp.float32), axis=0, keepdims=True)

    iou_at = jnp.sum(iou * gmask, axis=0, keepdims=True)
    xlab = jnp.sum(clsg * gmask, axis=0, keepdims=True)
    gtt = gtt_ref[0]
    gdims = (((1,), (0,)), ((), ()))
    g_hi = gtt.astype(jnp.bfloat16)
    g_lo = (gtt - g_hi.astype(jnp.float32)).astype(jnp.bfloat16)
    gm_b = gmask.astype(jnp.bfloat16)
    tcoord = (lax.dot_general(g_hi, gm_b, gdims,
                              preferred_element_type=jnp.float32)
              + lax.dot_general(g_lo, gm_b, gdims,
                                preferred_element_type=jnp.float32))
    tx1 = tcoord[0:1]
    tx2 = tcoord[1:2]
    ty1 = tcoord[2:3]
    ty2 = tcoord[3:4]

    nfg = jnp.maximum(jnp.sum(fg), 1.0)

    pwc = jnp.clip(px2 - px1, _EPS)
    phc = jnp.clip(py2 - py1, _EPS)
    twc = jnp.clip(tx2 - tx1, _EPS)
    thc = jnp.clip(ty2 - ty1, _EPS)
    iw = jnp.clip(jnp.minimum(px2, tx2) - jnp.maximum(px1, tx1), 0.0)
    ih = jnp.clip(jnp.minimum(py2, ty2) - jnp.maximum(py1, ty1), 0.0)
    inter2 = iw * ih
    union2 = pwc * phc + twc * thc - inter2 + _EPS
    iou2 = inter2 / union2
    cd = (((px1 + px2) * 0.5 - (tx1 + tx2) * 0.5) ** 2
          + ((py1 + py2) * 0.5 - (ty1 + ty2) * 0.5) ** 2)
    cw = jnp.maximum(px2, tx2) - jnp.minimum(px1, tx1)
    chh = jnp.maximum(py2, ty2) - jnp.minimum(py1, ty1)
    c2 = cw * cw + chh * chh + _EPS
    v = 4.0 / _PI2 * (_atan_pos(twc / thc) - _atan_pos(pwc / phc)) ** 2
    alpha = v / (v - iou2 + 1.0 + _EPS)
    ciou = iou2 - cd / c2 - alpha * v
    loss_box = jnp.sum(fg * (1.0 - ciou)) / nfg

    obj_t = fg * iou_at
    loss_obj = jnp.sum(_bce(obj, obj_t)) / float(_A)
    off = _CLS_SMOOTH / (_NUM_CLASSES - 1)
    scale = 1.0 - _CLS_SMOOTH - off
    e_abs = jnp.exp(-jnp.abs(cls))
    op1 = 1.0 + e_abs
    w1 = op1[0:40] * op1[40:80]
    w2 = w1[0:20] * w1[20:40]
    w3 = w2[0:10] * w2[10:20]
    sp_sum = (jnp.sum(jnp.maximum(cls, 0.0), axis=0, keepdims=True)
              + jnp.sum(jnp.log(w3), axis=0, keepdims=True))
    sumx = jnp.sum(cls, axis=0, keepdims=True)
    bce_mean = (sp_sum - off * sumx - scale * xlab) / float(_NUM_CLASSES)
    loss_cls = jnp.sum(fg * bce_mean) / nfg

    loss = (_LAMBDA_BOX * loss_box + _LAMBDA_OBJ * loss_obj
            + _LAMBDA_CLS * loss_cls)
    out_ref[...] = loss[None, None, None]


def kernel(p3, p4, p5, gt_boxes, gt_labels):
    b = p3.shape[0]
    p3f = p3.reshape(b, 5 + _NUM_CLASSES, _SIZES[0][0] * _SIZES[0][1])
    p4f = p4.reshape(b, 5 + _NUM_CLASSES, _SIZES[1][0] * _SIZES[1][1])
    p5f = p5.reshape(b, 5 + _NUM_CLASSES, _SIZES[2][0] * _SIZES[2][1])
    labs = gt_labels.astype(jnp.int32).reshape(b, 1, _G)
    cxg = gt_boxes[..., 0] * _IMG
    cyg = gt_boxes[..., 1] * _IMG
    wwg = gt_boxes[..., 2] * _IMG
    hhg = gt_boxes[..., 3] * _IMG
    gtt = jnp.stack([cxg - 0.5 * wwg, cxg + 0.5 * wwg,
                     cyg - 0.5 * hhg, cyg + 0.5 * hhg], axis=1)
    apx = jnp.asarray(_APX_NP)
    apy = jnp.asarray(_APY_NP)
    st = jnp.asarray(_ST_NP)
    ch = 5 + _NUM_CLASSES
    out = pl.pallas_call(
        _img_kernel,
        grid=(b,),
        in_specs=[
            pl.BlockSpec((1, ch, p3f.shape[2]), lambda i: (i, 0, 0)),
            pl.BlockSpec((1, ch, p4f.shape[2]), lambda i: (i, 0, 0)),
            pl.BlockSpec((1, ch, p5f.shape[2]), lambda i: (i, 0, 0)),
            pl.BlockSpec((1, _G, 4), lambda i: (i, 0, 0)),
            pl.BlockSpec((1, 4, _G), lambda i: (i, 0, 0)),
            pl.BlockSpec((1, 1, _G), lambda i: (i, 0, 0)),
            pl.BlockSpec((1, _A), lambda i: (0, 0)),
            pl.BlockSpec((1, _A), lambda i: (0, 0)),
            pl.BlockSpec((1, _A), lambda i: (0, 0)),
        ],
        out_specs=pl.BlockSpec((1, 1, 1), lambda i: (i, 0, 0)),
        out_shape=jax.ShapeDtypeStruct((b, 1, 1), jnp.float32),
        scratch_shapes=[pltpu.VMEM((2 * _G + 4, _A), jnp.float32)],
    )(p3f, p4f, p5f, gt_boxes, gtt, labs, apx, apy, st)
    return jnp.mean(out)


# speedup vs baseline: 1.0217x; 1.0174x over previous
import functools

import numpy as np
import jax
import jax.numpy as jnp
from jax import lax
from jax.experimental import pallas as pl
from jax.experimental.pallas import tpu as pltpu

_NUM_CLASSES = 80
_IMG = 512.0
_STRIDES = (8, 16, 32)
_SIZES = ((64, 64), (32, 32), (16, 16))
_LAMBDA_BOX = 5.0
_LAMBDA_OBJ = 1.0
_LAMBDA_CLS = 0.5
_ASSIGN_CLS_W = 0.5
_CENTER_RADIUS = 2.0
_TOPK = 20
_CLS_SMOOTH = 0.05
_AREA_MIN = 4.0 / 1.25
_AREA_MAX = 256.0 * 1.25
_SIZE_PRIOR_W = 0.2
_AR_PRIOR_W = 0.1
_IOU_COST_W = 3.0
_CENTER_COST_W = 0.5
_EPS = 1e-7
_PI2 = float(np.pi) ** 2
_A = sum(h * w for h, w in _SIZES)
_G = 30
_BIGI = np.int32(2 ** 30)


def _anchor_consts():
    apx, apy, st = [], [], []
    for (h, w), s in zip(_SIZES, _STRIDES):
        ys, xs = np.meshgrid(np.arange(h, dtype=np.float32),
                             np.arange(w, dtype=np.float32), indexing="ij")
        apx.append((xs.reshape(-1) + 0.5) * s)
        apy.append((ys.reshape(-1) + 0.5) * s)
        st.append(np.full((h * w,), float(s), dtype=np.float32))
    return (np.concatenate(apx)[None, :], np.concatenate(apy)[None, :],
            np.concatenate(st)[None, :])


_APX_NP, _APY_NP, _ST_NP = _anchor_consts()


def _sigmoid(x):
    return 1.0 / (1.0 + jnp.exp(-x))


def _softplus(x):
    return jnp.maximum(x, 0.0) + jnp.log1p(jnp.exp(-jnp.abs(x)))


def _bce(logits, targets):
    return (jnp.maximum(logits, 0.0) - logits * targets
            + jnp.log1p(jnp.exp(-jnp.abs(logits))))


def _atan_pos(x):
    big = x > 2.414213562373095
    mid = x > 0.4142135623730950
    xr = jnp.where(big, -1.0 / x, jnp.where(mid, (x - 1.0) / (x + 1.0), x))
    base = jnp.where(big, 0.5 * np.pi, jnp.where(mid, 0.25 * np.pi, 0.0))
    z = xr * xr
    p = 8.05374449538e-2
    p = p * z - 1.38776856032e-1
    p = p * z + 1.99777106478e-1
    p = p * z - 3.33329491539e-1
    return base + xr + xr * z * p


def _img_kernel(p3_ref, p4_ref, p5_ref, gtb_ref, gtt_ref, lab_ref,
                apx_ref, apy_ref, st_ref, out_ref, uni_ref):
    pred = jnp.concatenate([p3_ref[0], p4_ref[0], p5_ref[0]], axis=1)
    apx = apx_ref[...]
    apy = apy_ref[...]
    st = st_ref[...]

    obj = pred[4:5]
    cls = pred[5:5 + _NUM_CLASSES]

    cx = (2.0 * _sigmoid(pred[0:1]) - 1.0) * st + apx
    cy = (2.0 * _sigmoid(pred[1:2]) - 1.0) * st + apy
    pw = _softplus(pred[2:3]) * st
    ph = _softplus(pred[3:4]) * st
    px1 = cx - 0.5 * pw
    px2 = cx + 0.5 * pw
    py1 = cy - 0.5 * ph
    py2 = cy + 0.5 * ph

    gtb = gtb_ref[0]
    gcx = gtb[:, 0:1] * _IMG
    gcy = gtb[:, 1:2] * _IMG
    gww = gtb[:, 2:3] * _IMG
    ghh = gtb[:, 3:4] * _IMG
    gx1 = gcx - 0.5 * gww
    gx2 = gcx + 0.5 * gww
    gy1 = gcy - 0.5 * ghh
    gy2 = gcy + 0.5 * ghh
    gw = jnp.clip(gx2 - gx1, _EPS)
    gh = jnp.clip(gy2 - gy1, _EPS)

    inv_st2 = 1.0 / (st * st)
    area_cells = (gw * gh) * inv_st2
    gate = (area_cells >= _AREA_MIN) & (area_cells <= _AREA_MAX)
    dx = jnp.abs(apx - gcx)
    dy = jnp.abs(apy - gcy)
    r = _CENTER_RADIUS * st
    in_center = (dx < r) & (dy < r)
    in_box = (apx > gx1) & (apx < gx2) & (apy > gy1) & (apy < gy2)
    cand = gate & (in_center | in_box)

    a1 = jnp.clip(gx2 - gx1, 0.0) * jnp.clip(gy2 - gy1, 0.0)
    a2 = jnp.clip(px2 - px1, 0.0) * jnp.clip(py2 - py1, 0.0)
    ix1 = jnp.maximum(gx1, px1)
    iy1 = jnp.maximum(gy1, py1)
    ix2 = jnp.minimum(gx2, px2)
    iy2 = jnp.minimum(gy2, py2)
    inter = jnp.clip(ix2 - ix1, 0.0) * jnp.clip(iy2 - iy1, 0.0)
    iou = jnp.clip(inter / (a1 + a2 - inter + _EPS), 0.0, 1.0)

    pobj = _sigmoid(obj)
    t_exp = jnp.exp(-cls)
    sig_c = 1.0 / (1.0 + t_exp)
    omp = 1.0 - sig_c * pobj + _EPS
    y1 = omp[0:40] * omp[40:80]
    y2 = y1[0:20] * y1[20:40]
    y3 = y2[0:10] * y2[10:20]
    s_all = jnp.sum(jnp.log(y3), axis=0, keepdims=True)

    labs = lab_ref[0]
    c_iota = lax.broadcasted_iota(jnp.int32, (_NUM_CLASSES, _G), 0)
    oh_t = (labs == c_iota).astype(jnp.float32)
    dims = (((0,), (0,)), ((), ()))
    oh_b = oh_t.astype(jnp.bfloat16)
    c_hi = cls.astype(jnp.bfloat16)
    c_lo = (cls - c_hi.astype(jnp.float32)).astype(jnp.bfloat16)
    clsg = (lax.dot_general(oh_b, c_hi, dims,
                            preferred_element_type=jnp.float32)
            + lax.dot_general(oh_b, c_lo, dims,
                              preferred_element_type=jnp.float32))
    pg = _sigmoid(clsg) * pobj
    cls_sel = jnp.log(pg + _EPS) - jnp.log(1.0 - pg + _EPS)
    cls_cost = -(cls_sel + s_all)

    lpw = jnp.log(jnp.clip(px2 - px1, _EPS))
    lph = jnp.log(jnp.clip(py2 - py1, _EPS))
    lgw = jnp.log(gw)
    lgh = jnp.log(gh)
    size_cost = jnp.abs(lpw - lgw) + jnp.abs(lph - lgh)
    ar_cost = jnp.abs((lpw - lph) - (lgw - lgh))
    center_cost = jnp.sqrt(dx * dx + dy * dy) / st

    cost = (_IOU_COST_W * (1.0 - iou) + _ASSIGN_CLS_W * cls_cost
            + _SIZE_PRIOR_W * size_cost + _AR_PRIOR_W * ar_cost
            + _CENTER_COST_W * center_cost
            + jnp.where(cand, 0.0, 1e5))

    inf2 = jnp.full((2, _A), jnp.inf, jnp.float32)
    uni_ref[...] = jnp.concatenate(
        [-jnp.where(cand, iou, 0.0), inf2,
         jnp.where(cand, cost, jnp.inf), inf2], axis=0)
    cid20 = lax.broadcasted_iota(jnp.int32, (2 * _G + 4, _TOPK), 1)

    def ext_body(rr, mins):
        w = uni_ref[...]
        mn = jnp.min(w, axis=1, keepdims=True)
        uni_ref[...] = jnp.where(w == mn, jnp.inf, w)
        return jnp.where(cid20 == rr, mn, mins)

    mins = lax.fori_loop(0, _TOPK, ext_body,
                         jnp.full((2 * _G + 4, _TOPK), jnp.inf, jnp.float32))
    tsum = jnp.sum(jnp.maximum(-mins[0:_G], 0.0), axis=1, keepdims=True)
    dyn_k = jnp.clip(tsum.astype(jnp.int32), 1, _TOPK)
    j20 = lax.broadcasted_iota(jnp.int32, (_G, _TOPK), 1)
    tau = jnp.sum(jnp.where(j20 == dyn_k - 1, mins[_G + 2:2 * _G + 2], 0.0),
                  axis=1, keepdims=True)
    matched = cand & (cost <= tau)

    cost_m = jnp.where(matched, cost, jnp.inf)
    mn0 = jnp.min(cost_m, axis=0, keepdims=True)
    gmask = (cost_m == mn0).astype(jnp.float32)
    fg = jnp.max(matched.astype(jnp.float32), axis=0, keepdims=True)

    iou_at = jnp.sum(iou * gmask, axis=0, keepdims=True)
    xlab = jnp.sum(clsg * gmask, axis=0, keepdims=True)
    gtt = gtt_ref[0]
    gdims = (((1,), (0,)), ((), ()))
    g_hi = gtt.astype(jnp.bfloat16)
    g_lo = (gtt - g_hi.astype(jnp.float32)).astype(jnp.bfloat16)
    gm_b = gmask.astype(jnp.bfloat16)
    tcoord = (lax.dot_general(g_hi, gm_b, gdims,
                              preferred_element_type=jnp.float32)
              + lax.dot_general(g_lo, gm_b, gdims,
                                preferred_element_type=jnp.float32))
    tx1 = tcoord[0:1]
    tx2 = tcoord[1:2]
    ty1 = tcoord[2:3]
    ty2 = tcoord[3:4]

    nfg = jnp.maximum(jnp.sum(fg), 1.0)

    pwc = jnp.clip(px2 - px1, _EPS)
    phc = jnp.clip(py2 - py1, _EPS)
    twc = jnp.clip(tx2 - tx1, _EPS)
    thc = jnp.clip(ty2 - ty1, _EPS)
    iw = jnp.clip(jnp.minimum(px2, tx2) - jnp.maximum(px1, tx1), 0.0)
    ih = jnp.clip(jnp.minimum(py2, ty2) - jnp.maximum(py1, ty1), 0.0)
    inter2 = iw * ih
    union2 = pwc * phc + twc * thc - inter2 + _EPS
    iou2 = inter2 / union2
    cd = (((px1 + px2) * 0.5 - (tx1 + tx2) * 0.5) ** 2
          + ((py1 + py2) * 0.5 - (ty1 + ty2) * 0.5) ** 2)
    cw = jnp.maximum(px2, tx2) - jnp.minimum(px1, tx1)
    chh = jnp.maximum(py2, ty2) - jnp.minimum(py1, ty1)
    c2 = cw * cw + chh * chh + _EPS
    v = 4.0 / _PI2 * (_atan_pos(twc / thc) - _atan_pos(pwc / phc)) ** 2
    alpha = v / (v - iou2 + 1.0 + _EPS)
    ciou = iou2 - cd / c2 - alpha * v
    loss_box = jnp.sum(fg * (1.0 - ciou)) / nfg

    obj_t = fg * iou_at
    loss_obj = jnp.sum(_bce(obj, obj_t)) / float(_A)
    off = _CLS_SMOOTH / (_NUM_CLASSES - 1)
    scale = 1.0 - _CLS_SMOOTH - off
    e_abs = jnp.exp(-jnp.abs(cls))
    op1 = 1.0 + e_abs
    w1 = op1[0:40] * op1[40:80]
    w2 = w1[0:20] * w1[20:40]
    w3 = w2[0:10] * w2[10:20]
    sp_sum = (jnp.sum(jnp.maximum(cls, 0.0), axis=0, keepdims=True)
              + jnp.sum(jnp.log(w3), axis=0, keepdims=True))
    sumx = jnp.sum(cls, axis=0, keepdims=True)
    bce_mean = (sp_sum - off * sumx - scale * xlab) / float(_NUM_CLASSES)
    loss_cls = jnp.sum(fg * bce_mean) / nfg

    loss = (_LAMBDA_BOX * loss_box + _LAMBDA_OBJ * loss_obj
            + _LAMBDA_CLS * loss_cls)
    out_ref[...] = loss[None, None, None]


def kernel(p3, p4, p5, gt_boxes, gt_labels):
    b = p3.shape[0]
    p3f = p3.reshape(b, 5 + _NUM_CLASSES, _SIZES[0][0] * _SIZES[0][1])
    p4f = p4.reshape(b, 5 + _NUM_CLASSES, _SIZES[1][0] * _SIZES[1][1])
    p5f = p5.reshape(b, 5 + _NUM_CLASSES, _SIZES[2][0] * _SIZES[2][1])
    labs = gt_labels.astype(jnp.int32).reshape(b, 1, _G)
    cxg = gt_boxes[..., 0] * _IMG
    cyg = gt_boxes[..., 1] * _IMG
    wwg = gt_boxes[..., 2] * _IMG
    hhg = gt_boxes[..., 3] * _IMG
    gtt = jnp.stack([cxg - 0.5 * wwg, cxg + 0.5 * wwg,
                     cyg - 0.5 * hhg, cyg + 0.5 * hhg], axis=1)
    apx = jnp.asarray(_APX_NP)
    apy = jnp.asarray(_APY_NP)
    st = jnp.asarray(_ST_NP)
    ch = 5 + _NUM_CLASSES
    out = pl.pallas_call(
        _img_kernel,
        grid=(b,),
        in_specs=[
            pl.BlockSpec((1, ch, p3f.shape[2]), lambda i: (i, 0, 0)),
            pl.BlockSpec((1, ch, p4f.shape[2]), lambda i: (i, 0, 0)),
            pl.BlockSpec((1, ch, p5f.shape[2]), lambda i: (i, 0, 0)),
            pl.BlockSpec((1, _G, 4), lambda i: (i, 0, 0)),
            pl.BlockSpec((1, 4, _G), lambda i: (i, 0, 0)),
            pl.BlockSpec((1, 1, _G), lambda i: (i, 0, 0)),
            pl.BlockSpec((1, _A), lambda i: (0, 0)),
            pl.BlockSpec((1, _A), lambda i: (0, 0)),
            pl.BlockSpec((1, _A), lambda i: (0, 0)),
        ],
        out_specs=pl.BlockSpec((1, 1, 1), lambda i: (i, 0, 0)),
        out_shape=jax.ShapeDtypeStruct((b, 1, 1), jnp.float32),
        scratch_shapes=[pltpu.VMEM((2 * _G + 4, _A), jnp.float32)],
    )(p3f, p4f, p5f, gt_boxes, gtt, labs, apx, apy, st)
    return jnp.mean(out)


# speedup vs baseline: 1.0369x; 1.0148x over previous
import functools

import numpy as np
import jax
import jax.numpy as jnp
from jax import lax
from jax.experimental import pallas as pl
from jax.experimental.pallas import tpu as pltpu

_NUM_CLASSES = 80
_IMG = 512.0
_STRIDES = (8, 16, 32)
_SIZES = ((64, 64), (32, 32), (16, 16))
_LAMBDA_BOX = 5.0
_LAMBDA_OBJ = 1.0
_LAMBDA_CLS = 0.5
_ASSIGN_CLS_W = 0.5
_CENTER_RADIUS = 2.0
_TOPK = 20
_CLS_SMOOTH = 0.05
_AREA_MIN = 4.0 / 1.25
_AREA_MAX = 256.0 * 1.25
_SIZE_PRIOR_W = 0.2
_AR_PRIOR_W = 0.1
_IOU_COST_W = 3.0
_CENTER_COST_W = 0.5
_EPS = 1e-7
_PI2 = float(np.pi) ** 2
_A = sum(h * w for h, w in _SIZES)
_G = 30
_BIGI = np.int32(2 ** 30)


def _anchor_consts():
    apx, apy, st = [], [], []
    for (h, w), s in zip(_SIZES, _STRIDES):
        ys, xs = np.meshgrid(np.arange(h, dtype=np.float32),
                             np.arange(w, dtype=np.float32), indexing="ij")
        apx.append((xs.reshape(-1) + 0.5) * s)
        apy.append((ys.reshape(-1) + 0.5) * s)
        st.append(np.full((h * w,), float(s), dtype=np.float32))
    return (np.concatenate(apx)[None, :], np.concatenate(apy)[None, :],
            np.concatenate(st)[None, :])


_APX_NP, _APY_NP, _ST_NP = _anchor_consts()


def _sigmoid(x):
    return 1.0 / (1.0 + jnp.exp(-x))


def _softplus(x):
    return jnp.maximum(x, 0.0) + jnp.log1p(jnp.exp(-jnp.abs(x)))


def _bce(logits, targets):
    return (jnp.maximum(logits, 0.0) - logits * targets
            + jnp.log1p(jnp.exp(-jnp.abs(logits))))


def _atan_pos(x):
    big = x > 2.414213562373095
    mid = x > 0.4142135623730950
    xr = jnp.where(big, -1.0 / x, jnp.where(mid, (x - 1.0) / (x + 1.0), x))
    base = jnp.where(big, 0.5 * np.pi, jnp.where(mid, 0.25 * np.pi, 0.0))
    z = xr * xr
    p = 8.05374449538e-2
    p = p * z - 1.38776856032e-1
    p = p * z + 1.99777106478e-1
    p = p * z - 3.33329491539e-1
    return base + xr + xr * z * p


def _img_kernel(p3_ref, p4_ref, p5_ref, gtb_ref, gtt_ref, lab_ref,
                apx_ref, apy_ref, st_ref, out_ref, tk_ref, sel_ref):
    pred = jnp.concatenate([p3_ref[0], p4_ref[0], p5_ref[0]], axis=1)
    apx = apx_ref[...]
    apy = apy_ref[...]
    st = st_ref[...]

    obj = pred[4:5]
    cls = pred[5:5 + _NUM_CLASSES]

    cx = (2.0 * _sigmoid(pred[0:1]) - 1.0) * st + apx
    cy = (2.0 * _sigmoid(pred[1:2]) - 1.0) * st + apy
    pw = _softplus(pred[2:3]) * st
    ph = _softplus(pred[3:4]) * st
    px1 = cx - 0.5 * pw
    px2 = cx + 0.5 * pw
    py1 = cy - 0.5 * ph
    py2 = cy + 0.5 * ph

    gtb = gtb_ref[0]
    gcx = gtb[:, 0:1] * _IMG
    gcy = gtb[:, 1:2] * _IMG
    gww = gtb[:, 2:3] * _IMG
    ghh = gtb[:, 3:4] * _IMG
    gx1 = gcx - 0.5 * gww
    gx2 = gcx + 0.5 * gww
    gy1 = gcy - 0.5 * ghh
    gy2 = gcy + 0.5 * ghh
    gw = jnp.clip(gx2 - gx1, _EPS)
    gh = jnp.clip(gy2 - gy1, _EPS)

    inv_st2 = 1.0 / (st * st)
    area_cells = (gw * gh) * inv_st2
    gate = (area_cells >= _AREA_MIN) & (area_cells <= _AREA_MAX)
    dx = jnp.abs(apx - gcx)
    dy = jnp.abs(apy - gcy)
    r = _CENTER_RADIUS * st
    in_center = (dx < r) & (dy < r)
    in_box = (apx > gx1) & (apx < gx2) & (apy > gy1) & (apy < gy2)
    cand = gate & (in_center | in_box)

    a1 = jnp.clip(gx2 - gx1, 0.0) * jnp.clip(gy2 - gy1, 0.0)
    a2 = jnp.clip(px2 - px1, 0.0) * jnp.clip(py2 - py1, 0.0)
    ix1 = jnp.maximum(gx1, px1)
    iy1 = jnp.maximum(gy1, py1)
    ix2 = jnp.minimum(gx2, px2)
    iy2 = jnp.minimum(gy2, py2)
    inter = jnp.clip(ix2 - ix1, 0.0) * jnp.clip(iy2 - iy1, 0.0)
    iou = jnp.clip(inter / (a1 + a2 - inter + _EPS), 0.0, 1.0)

    pobj = _sigmoid(obj)
    t_exp = jnp.exp(-cls)
    sig_c = 1.0 / (1.0 + t_exp)
    omp = 1.0 - sig_c * pobj + _EPS
    y1 = omp[0:40] * omp[40:80]
    y2 = y1[0:20] * y1[20:40]
    y3 = y2[0:10] * y2[10:20]
    s_all = jnp.sum(jnp.log(y3), axis=0, keepdims=True)

    labs = lab_ref[0]
    c_iota = lax.broadcasted_iota(jnp.int32, (_NUM_CLASSES, _G), 0)
    oh_t = (labs == c_iota).astype(jnp.float32)
    dims = (((0,), (0,)), ((), ()))
    oh_b = oh_t.astype(jnp.bfloat16)
    c_hi = cls.astype(jnp.bfloat16)
    c_lo = (cls - c_hi.astype(jnp.float32)).astype(jnp.bfloat16)
    clsg = (lax.dot_general(oh_b, c_hi, dims,
                            preferred_element_type=jnp.float32)
            + lax.dot_general(oh_b, c_lo, dims,
                              preferred_element_type=jnp.float32))
    pg = _sigmoid(clsg) * pobj
    cls_sel = jnp.log(pg + _EPS) - jnp.log(1.0 - pg + _EPS)
    cls_cost = -(cls_sel + s_all)

    lpw = jnp.log(jnp.clip(px2 - px1, _EPS))
    lph = jnp.log(jnp.clip(py2 - py1, _EPS))
    lgw = jnp.log(gw)
    lgh = jnp.log(gh)
    size_cost = jnp.abs(lpw - lgw) + jnp.abs(lph - lgh)
    ar_cost = jnp.abs((lpw - lph) - (lgw - lgh))
    center_cost = jnp.sqrt(dx * dx + dy * dy) / st

    cost = (_IOU_COST_W * (1.0 - iou) + _ASSIGN_CLS_W * cls_cost
            + _SIZE_PRIOR_W * size_cost + _AR_PRIOR_W * ar_cost
            + _CENTER_COST_W * center_cost
            + jnp.where(cand, 0.0, 1e5))

    tk_ref[...] = -jnp.where(cand, iou, 0.0)
    sel_ref[...] = jnp.where(cand, cost, jnp.inf)
    cid20 = lax.broadcasted_iota(jnp.int32, (_G, _TOPK), 1)
    half = _A // 2

    def ext_body(rr, carry):
        tmins, smins = carry
        tw = tk_ref[...]
        mnt = jnp.min(jnp.minimum(tw[:, :half], tw[:, half:]), axis=1,
                      keepdims=True)
        tk_ref[...] = jnp.where(tw == mnt, jnp.inf, tw)
        sw = sel_ref[...]
        mns = jnp.min(jnp.minimum(sw[:, :half], sw[:, half:]), axis=1,
                      keepdims=True)
        sel_ref[...] = jnp.where(sw == mns, jnp.inf, sw)
        return (jnp.where(cid20 == rr, mnt, tmins),
                jnp.where(cid20 == rr, mns, smins))

    minf = jnp.full((_G, _TOPK), jnp.inf, jnp.float32)
    tmins, smins = lax.fori_loop(0, _TOPK, ext_body, (minf, minf))
    tsum = jnp.sum(jnp.maximum(-tmins, 0.0), axis=1, keepdims=True)
    dyn_k = jnp.clip(tsum.astype(jnp.int32), 1, _TOPK)
    j20 = lax.broadcasted_iota(jnp.int32, (_G, _TOPK), 1)
    tau = jnp.sum(jnp.where(j20 == dyn_k - 1, smins, 0.0),
                  axis=1, keepdims=True)
    matched = cand & (cost <= tau)

    cost_m = jnp.where(matched, cost, jnp.inf)
    mn0 = jnp.min(cost_m, axis=0, keepdims=True)
    gmask = (cost_m == mn0).astype(jnp.float32)
    fg = jnp.max(matched.astype(jnp.float32), axis=0, keepdims=True)

    iou_at = jnp.sum(iou * gmask, axis=0, keepdims=True)
    xlab = jnp.sum(clsg * gmask, axis=0, keepdims=True)
    gtt = gtt_ref[0]
    gdims = (((1,), (0,)), ((), ()))
    g_hi = gtt.astype(jnp.bfloat16)
    g_lo = (gtt - g_hi.astype(jnp.float32)).astype(jnp.bfloat16)
    gm_b = gmask.astype(jnp.bfloat16)
    tcoord = (lax.dot_general(g_hi, gm_b, gdims,
                              preferred_element_type=jnp.float32)
              + lax.dot_general(g_lo, gm_b, gdims,
                                preferred_element_type=jnp.float32))
    tx1 = tcoord[0:1]
    tx2 = tcoord[1:2]
    ty1 = tcoord[2:3]
    ty2 = tcoord[3:4]

    nfg = jnp.maximum(jnp.sum(fg), 1.0)

    pwc = jnp.clip(px2 - px1, _EPS)
    phc = jnp.clip(py2 - py1, _EPS)
    twc = jnp.clip(tx2 - tx1, _EPS)
    thc = jnp.clip(ty2 - ty1, _EPS)
    iw = jnp.clip(jnp.minimum(px2, tx2) - jnp.maximum(px1, tx1), 0.0)
    ih = jnp.clip(jnp.minimum(py2, ty2) - jnp.maximum(py1, ty1), 0.0)
    inter2 = iw * ih
    union2 = pwc * phc + twc * thc - inter2 + _EPS
    iou2 = inter2 / union2
    cd = (((px1 + px2) * 0.5 - (tx1 + tx2) * 0.5) ** 2
          + ((py1 + py2) * 0.5 - (ty1 + ty2) * 0.5) ** 2)
    cw = jnp.maximum(px2, tx2) - jnp.minimum(px1, tx1)
    chh = jnp.maximum(py2, ty2) - jnp.minimum(py1, ty1)
    c2 = cw * cw + chh * chh + _EPS
    v = 4.0 / _PI2 * (_atan_pos(twc / thc) - _atan_pos(pwc / phc)) ** 2
    alpha = v / (v - iou2 + 1.0 + _EPS)
    ciou = iou2 - cd / c2 - alpha * v
    loss_box = jnp.sum(fg * (1.0 - ciou)) / nfg

    obj_t = fg * iou_at
    loss_obj = jnp.sum(_bce(obj, obj_t)) / float(_A)
    off = _CLS_SMOOTH / (_NUM_CLASSES - 1)
    scale = 1.0 - _CLS_SMOOTH - off
    e_abs = jnp.exp(-jnp.abs(cls))
    op1 = 1.0 + e_abs
    w1 = op1[0:40] * op1[40:80]
    w2 = w1[0:20] * w1[20:40]
    w3 = w2[0:10] * w2[10:20]
    sp_sum = (jnp.sum(jnp.maximum(cls, 0.0), axis=0, keepdims=True)
              + jnp.sum(jnp.log(w3), axis=0, keepdims=True))
    sumx = jnp.sum(cls, axis=0, keepdims=True)
    bce_mean = (sp_sum - off * sumx - scale * xlab) / float(_NUM_CLASSES)
    loss_cls = jnp.sum(fg * bce_mean) / nfg

    loss = (_LAMBDA_BOX * loss_box + _LAMBDA_OBJ * loss_obj
            + _LAMBDA_CLS * loss_cls)
    out_ref[...] = loss[None, None, None]


def kernel(p3, p4, p5, gt_boxes, gt_labels):
    b = p3.shape[0]
    p3f = p3.reshape(b, 5 + _NUM_CLASSES, _SIZES[0][0] * _SIZES[0][1])
    p4f = p4.reshape(b, 5 + _NUM_CLASSES, _SIZES[1][0] * _SIZES[1][1])
    p5f = p5.reshape(b, 5 + _NUM_CLASSES, _SIZES[2][0] * _SIZES[2][1])
    labs = gt_labels.astype(jnp.int32).reshape(b, 1, _G)
    cxg = gt_boxes[..., 0] * _IMG
    cyg = gt_boxes[..., 1] * _IMG
    wwg = gt_boxes[..., 2] * _IMG
    hhg = gt_boxes[..., 3] * _IMG
    gtt = jnp.stack([cxg - 0.5 * wwg, cxg + 0.5 * wwg,
                     cyg - 0.5 * hhg, cyg + 0.5 * hhg], axis=1)
    apx = jnp.asarray(_APX_NP)
    apy = jnp.asarray(_APY_NP)
    st = jnp.asarray(_ST_NP)
    ch = 5 + _NUM_CLASSES
    out = pl.pallas_call(
        _img_kernel,
        grid=(b,),
        in_specs=[
            pl.BlockSpec((1, ch, p3f.shape[2]), lambda i: (i, 0, 0)),
            pl.BlockSpec((1, ch, p4f.shape[2]), lambda i: (i, 0, 0)),
            pl.BlockSpec((1, ch, p5f.shape[2]), lambda i: (i, 0, 0)),
            pl.BlockSpec((1, _G, 4), lambda i: (i, 0, 0)),
            pl.BlockSpec((1, 4, _G), lambda i: (i, 0, 0)),
            pl.BlockSpec((1, 1, _G), lambda i: (i, 0, 0)),
            pl.BlockSpec((1, _A), lambda i: (0, 0)),
            pl.BlockSpec((1, _A), lambda i: (0, 0)),
            pl.BlockSpec((1, _A), lambda i: (0, 0)),
        ],
        out_specs=pl.BlockSpec((1, 1, 1), lambda i: (i, 0, 0)),
        out_shape=jax.ShapeDtypeStruct((b, 1, 1), jnp.float32),
        scratch_shapes=[pltpu.VMEM((_G, _A), jnp.float32),
                        pltpu.VMEM((_G, _A), jnp.float32)],
    )(p3f, p4f, p5f, gt_boxes, gtt, labs, apx, apy, st)
    return jnp.mean(out)


# speedup vs baseline: 1.0471x; 1.0099x over previous
import functools

import numpy as np
import jax
import jax.numpy as jnp
from jax import lax
from jax.experimental import pallas as pl
from jax.experimental.pallas import tpu as pltpu

_NUM_CLASSES = 80
_IMG = 512.0
_STRIDES = (8, 16, 32)
_SIZES = ((64, 64), (32, 32), (16, 16))
_LAMBDA_BOX = 5.0
_LAMBDA_OBJ = 1.0
_LAMBDA_CLS = 0.5
_ASSIGN_CLS_W = 0.5
_CENTER_RADIUS = 2.0
_TOPK = 20
_CLS_SMOOTH = 0.05
_AREA_MIN = 4.0 / 1.25
_AREA_MAX = 256.0 * 1.25
_SIZE_PRIOR_W = 0.2
_AR_PRIOR_W = 0.1
_IOU_COST_W = 3.0
_CENTER_COST_W = 0.5
_EPS = 1e-7
_PI2 = float(np.pi) ** 2
_A = sum(h * w for h, w in _SIZES)
_G = 30
_BIGI = np.int32(2 ** 30)


def _anchor_consts():
    apx, apy, st = [], [], []
    for (h, w), s in zip(_SIZES, _STRIDES):
        ys, xs = np.meshgrid(np.arange(h, dtype=np.float32),
                             np.arange(w, dtype=np.float32), indexing="ij")
        apx.append((xs.reshape(-1) + 0.5) * s)
        apy.append((ys.reshape(-1) + 0.5) * s)
        st.append(np.full((h * w,), float(s), dtype=np.float32))
    return (np.concatenate(apx)[None, :], np.concatenate(apy)[None, :],
            np.concatenate(st)[None, :])


_APX_NP, _APY_NP, _ST_NP = _anchor_consts()


def _sigmoid(x):
    return 1.0 / (1.0 + jnp.exp(-x))


def _softplus(x):
    return jnp.maximum(x, 0.0) + jnp.log1p(jnp.exp(-jnp.abs(x)))


def _bce(logits, targets):
    return (jnp.maximum(logits, 0.0) - logits * targets
            + jnp.log1p(jnp.exp(-jnp.abs(logits))))


def _atan_pos(x):
    big = x > 2.414213562373095
    mid = x > 0.4142135623730950
    xr = jnp.where(big, -1.0 / x, jnp.where(mid, (x - 1.0) / (x + 1.0), x))
    base = jnp.where(big, 0.5 * np.pi, jnp.where(mid, 0.25 * np.pi, 0.0))
    z = xr * xr
    p = 8.05374449538e-2
    p = p * z - 1.38776856032e-1
    p = p * z + 1.99777106478e-1
    p = p * z - 3.33329491539e-1
    return base + xr + xr * z * p


def _img_kernel(p3_ref, p4_ref, p5_ref, gtb_ref, gtt_ref, lab_ref,
                apx_ref, apy_ref, st_ref, out_ref, tk_ref, sel_ref):
    pred = jnp.concatenate([p3_ref[0], p4_ref[0], p5_ref[0]], axis=1)
    apx = apx_ref[...]
    apy = apy_ref[...]
    st = st_ref[...]

    obj = pred[4:5]
    cls = pred[5:5 + _NUM_CLASSES]

    cx = (2.0 * _sigmoid(pred[0:1]) - 1.0) * st + apx
    cy = (2.0 * _sigmoid(pred[1:2]) - 1.0) * st + apy
    pw = _softplus(pred[2:3]) * st
    ph = _softplus(pred[3:4]) * st
    px1 = cx - 0.5 * pw
    px2 = cx + 0.5 * pw
    py1 = cy - 0.5 * ph
    py2 = cy + 0.5 * ph

    gtb = gtb_ref[0]
    gcx = gtb[:, 0:1] * _IMG
    gcy = gtb[:, 1:2] * _IMG
    gww = gtb[:, 2:3] * _IMG
    ghh = gtb[:, 3:4] * _IMG
    gx1 = gcx - 0.5 * gww
    gx2 = gcx + 0.5 * gww
    gy1 = gcy - 0.5 * ghh
    gy2 = gcy + 0.5 * ghh
    gw = jnp.clip(gx2 - gx1, _EPS)
    gh = jnp.clip(gy2 - gy1, _EPS)

    inv_st2 = 1.0 / (st * st)
    area_cells = (gw * gh) * inv_st2
    gate = (area_cells >= _AREA_MIN) & (area_cells <= _AREA_MAX)
    dx = jnp.abs(apx - gcx)
    dy = jnp.abs(apy - gcy)
    r = _CENTER_RADIUS * st
    in_center = (dx < r) & (dy < r)
    in_box = (apx > gx1) & (apx < gx2) & (apy > gy1) & (apy < gy2)
    cand = gate & (in_center | in_box)

    a1 = jnp.clip(gx2 - gx1, 0.0) * jnp.clip(gy2 - gy1, 0.0)
    a2 = jnp.clip(px2 - px1, 0.0) * jnp.clip(py2 - py1, 0.0)
    ix1 = jnp.maximum(gx1, px1)
    iy1 = jnp.maximum(gy1, py1)
    ix2 = jnp.minimum(gx2, px2)
    iy2 = jnp.minimum(gy2, py2)
    inter = jnp.clip(ix2 - ix1, 0.0) * jnp.clip(iy2 - iy1, 0.0)
    iou = jnp.clip(inter / (a1 + a2 - inter + _EPS), 0.0, 1.0)

    pobj = _sigmoid(obj)
    t_exp = jnp.exp(-cls)
    sig_c = 1.0 / (1.0 + t_exp)
    omp = 1.0 - sig_c * pobj + _EPS
    y1 = omp[0:40] * omp[40:80]
    y2 = y1[0:20] * y1[20:40]
    y3 = y2[0:10] * y2[10:20]
    s_all = jnp.sum(jnp.log(y3), axis=0, keepdims=True)

    labs = lab_ref[0]
    c_iota = lax.broadcasted_iota(jnp.int32, (_NUM_CLASSES, _G), 0)
    oh_t = (labs == c_iota).astype(jnp.float32)
    dims = (((0,), (0,)), ((), ()))
    oh_b = oh_t.astype(jnp.bfloat16)
    c_hi = cls.astype(jnp.bfloat16)
    c_lo = (cls - c_hi.astype(jnp.float32)).astype(jnp.bfloat16)
    clsg = (lax.dot_general(oh_b, c_hi, dims,
                            preferred_element_type=jnp.float32)
            + lax.dot_general(oh_b, c_lo, dims,
                              preferred_element_type=jnp.float32))
    pg = _sigmoid(clsg) * pobj
    cls_sel = jnp.log(pg + _EPS) - jnp.log(1.0 - pg + _EPS)
    cls_cost = -(cls_sel + s_all)

    lpw = jnp.log(jnp.clip(px2 - px1, _EPS))
    lph = jnp.log(jnp.clip(py2 - py1, _EPS))
    lgw = jnp.log(gw)
    lgh = jnp.log(gh)
    size_cost = jnp.abs(lpw - lgw) + jnp.abs(lph - lgh)
    ar_cost = jnp.abs((lpw - lph) - (lgw - lgh))
    center_cost = jnp.sqrt(dx * dx + dy * dy) / st

    cost = (_IOU_COST_W * (1.0 - iou) + _ASSIGN_CLS_W * cls_cost
            + _SIZE_PRIOR_W * size_cost + _AR_PRIOR_W * ar_cost
            + _CENTER_COST_W * center_cost
            + jnp.where(cand, 0.0, 1e5))

    tk_ref[...] = -jnp.where(cand, iou, 0.0)
    sel_ref[...] = jnp.where(cand, cost, jnp.inf)
    cid20 = lax.broadcasted_iota(jnp.int32, (_G, _TOPK), 1)
    half = _A // 2

    def ext_step(rr, tmins, smins):
        tw = tk_ref[...]
        mnt = jnp.min(jnp.minimum(tw[:, :half], tw[:, half:]), axis=1,
                      keepdims=True)
        tk_ref[...] = jnp.where(tw == mnt, jnp.inf, tw)
        sw = sel_ref[...]
        mns = jnp.min(jnp.minimum(sw[:, :half], sw[:, half:]), axis=1,
                      keepdims=True)
        sel_ref[...] = jnp.where(sw == mns, jnp.inf, sw)
        return (jnp.where(cid20 == rr, mnt, tmins),
                jnp.where(cid20 == rr, mns, smins))

    def ext_body(it, carry):
        tmins, smins = ext_step(2 * it, *carry)
        return ext_step(2 * it + 1, tmins, smins)

    minf = jnp.full((_G, _TOPK), jnp.inf, jnp.float32)
    tmins, smins = lax.fori_loop(0, _TOPK // 2, ext_body, (minf, minf))
    tsum = jnp.sum(jnp.maximum(-tmins, 0.0), axis=1, keepdims=True)
    dyn_k = jnp.clip(tsum.astype(jnp.int32), 1, _TOPK)
    j20 = lax.broadcasted_iota(jnp.int32, (_G, _TOPK), 1)
    tau = jnp.sum(jnp.where(j20 == dyn_k - 1, smins, 0.0),
                  axis=1, keepdims=True)
    matched = cand & (cost <= tau)

    cost_m = jnp.where(matched, cost, jnp.inf)
    mn0 = jnp.min(cost_m, axis=0, keepdims=True)
    gmask = (cost_m == mn0).astype(jnp.float32)
    fg = jnp.max(matched.astype(jnp.float32), axis=0, keepdims=True)

    iou_at = jnp.sum(iou * gmask, axis=0, keepdims=True)
    xlab = jnp.sum(clsg * gmask, axis=0, keepdims=True)
    gtt = gtt_ref[0]
    gdims = (((1,), (0,)), ((), ()))
    g_hi = gtt.astype(jnp.bfloat16)
    g_lo = (gtt - g_hi.astype(jnp.float32)).astype(jnp.bfloat16)
    gm_b = gmask.astype(jnp.bfloat16)
    tcoord = (lax.dot_general(g_hi, gm_b, gdims,
                              preferred_element_type=jnp.float32)
              + lax.dot_general(g_lo, gm_b, gdims,
                                preferred_element_type=jnp.float32))
    tx1 = tcoord[0:1]
    tx2 = tcoord[1:2]
    ty1 = tcoord[2:3]
    ty2 = tcoord[3:4]

    nfg = jnp.maximum(jnp.sum(fg), 1.0)

    pwc = jnp.clip(px2 - px1, _EPS)
    phc = jnp.clip(py2 - py1, _EPS)
    twc = jnp.clip(tx2 - tx1, _EPS)
    thc = jnp.clip(ty2 - ty1, _EPS)
    iw = jnp.clip(jnp.minimum(px2, tx2) - jnp.maximum(px1, tx1), 0.0)
    ih = jnp.clip(jnp.minimum(py2, ty2) - jnp.maximum(py1, ty1), 0.0)
    inter2 = iw * ih
    union2 = pwc * phc + twc * thc - inter2 + _EPS
    iou2 = inter2 / union2
    cd = (((px1 + px2) * 0.5 - (tx1 + tx2) * 0.5) ** 2
          + ((py1 + py2) * 0.5 - (ty1 + ty2) * 0.5) ** 2)
    cw = jnp.maximum(px2, tx2) - jnp.minimum(px1, tx1)
    chh = jnp.maximum(py2, ty2) - jnp.minimum(py1, ty1)
    c2 = cw * cw + chh * chh + _EPS
    v = 4.0 / _PI2 * (_atan_pos(twc / thc) - _atan_pos(pwc / phc)) ** 2
    alpha = v / (v - iou2 + 1.0 + _EPS)
    ciou = iou2 - cd / c2 - alpha * v
    loss_box = jnp.sum(fg * (1.0 - ciou)) / nfg

    obj_t = fg * iou_at
    loss_obj = jnp.sum(_bce(obj, obj_t)) / float(_A)
    off = _CLS_SMOOTH / (_NUM_CLASSES - 1)
    scale = 1.0 - _CLS_SMOOTH - off
    e_abs = jnp.exp(-jnp.abs(cls))
    op1 = 1.0 + e_abs
    w1 = op1[0:40] * op1[40:80]
    w2 = w1[0:20] * w1[20:40]
    w3 = w2[0:10] * w2[10:20]
    sp_sum = (jnp.sum(jnp.maximum(cls, 0.0), axis=0, keepdims=True)
              + jnp.sum(jnp.log(w3), axis=0, keepdims=True))
    sumx = jnp.sum(cls, axis=0, keepdims=True)
    bce_mean = (sp_sum - off * sumx - scale * xlab) / float(_NUM_CLASSES)
    loss_cls = jnp.sum(fg * bce_mean) / nfg

    loss = (_LAMBDA_BOX * loss_box + _LAMBDA_OBJ * loss_obj
            + _LAMBDA_CLS * loss_cls)
    out_ref[...] = loss[None, None, None]


def kernel(p3, p4, p5, gt_boxes, gt_labels):
    b = p3.shape[0]
    p3f = p3.reshape(b, 5 + _NUM_CLASSES, _SIZES[0][0] * _SIZES[0][1])
    p4f = p4.reshape(b, 5 + _NUM_CLASSES, _SIZES[1][0] * _SIZES[1][1])
    p5f = p5.reshape(b, 5 + _NUM_CLASSES, _SIZES[2][0] * _SIZES[2][1])
    labs = gt_labels.astype(jnp.int32).reshape(b, 1, _G)
    cxg = gt_boxes[..., 0] * _IMG
    cyg = gt_boxes[..., 1] * _IMG
    wwg = gt_boxes[..., 2] * _IMG
    hhg = gt_boxes[..., 3] * _IMG
    gtt = jnp.stack([cxg - 0.5 * wwg, cxg + 0.5 * wwg,
                     cyg - 0.5 * hhg, cyg + 0.5 * hhg], axis=1)
    apx = jnp.asarray(_APX_NP)
    apy = jnp.asarray(_APY_NP)
    st = jnp.asarray(_ST_NP)
    ch = 5 + _NUM_CLASSES
    out = pl.pallas_call(
        _img_kernel,
        grid=(b,),
        in_specs=[
            pl.BlockSpec((1, ch, p3f.shape[2]), lambda i: (i, 0, 0)),
            pl.BlockSpec((1, ch, p4f.shape[2]), lambda i: (i, 0, 0)),
            pl.BlockSpec((1, ch, p5f.shape[2]), lambda i: (i, 0, 0)),
            pl.BlockSpec((1, _G, 4), lambda i: (i, 0, 0)),
            pl.BlockSpec((1, 4, _G), lambda i: (i, 0, 0)),
            pl.BlockSpec((1, 1, _G), lambda i: (i, 0, 0)),
            pl.BlockSpec((1, _A), lambda i: (0, 0)),
            pl.BlockSpec((1, _A), lambda i: (0, 0)),
            pl.BlockSpec((1, _A), lambda i: (0, 0)),
        ],
        out_specs=pl.BlockSpec((1, 1, 1), lambda i: (i, 0, 0)),
        out_shape=jax.ShapeDtypeStruct((b, 1, 1), jnp.float32),
        scratch_shapes=[pltpu.VMEM((_G, _A), jnp.float32),
                        pltpu.VMEM((_G, _A), jnp.float32)],
    )(p3f, p4f, p5f, gt_boxes, gtt, labs, apx, apy, st)
    return jnp.mean(out)


# speedup vs baseline: 1.0482x; 1.0010x over previous
import numpy as np
import jax
import jax.numpy as jnp
from jax import lax
from jax.experimental import pallas as pl
from jax.experimental.pallas import tpu as pltpu

_NUM_CLASSES = 80
_IMG = 512.0
_STRIDES = (8, 16, 32)
_SIZES = ((64, 64), (32, 32), (16, 16))
_LAMBDA_BOX = 5.0
_LAMBDA_OBJ = 1.0
_LAMBDA_CLS = 0.5
_ASSIGN_CLS_W = 0.5
_CENTER_RADIUS = 2.0
_TOPK = 20
_CLS_SMOOTH = 0.05
_AREA_MIN = 4.0 / 1.25
_AREA_MAX = 256.0 * 1.25
_SIZE_PRIOR_W = 0.2
_AR_PRIOR_W = 0.1
_IOU_COST_W = 3.0
_CENTER_COST_W = 0.5
_EPS = 1e-7
_PI2 = float(np.pi) ** 2
_A = sum(h * w for h, w in _SIZES)
_G = 30


def _anchor_consts():
    apx, apy, st = [], [], []
    for (h, w), s in zip(_SIZES, _STRIDES):
        ys, xs = np.meshgrid(np.arange(h, dtype=np.float32),
                             np.arange(w, dtype=np.float32), indexing="ij")
        apx.append((xs.reshape(-1) + 0.5) * s)
        apy.append((ys.reshape(-1) + 0.5) * s)
        st.append(np.full((h * w,), float(s), dtype=np.float32))
    return (np.concatenate(apx)[None, :], np.concatenate(apy)[None, :],
            np.concatenate(st)[None, :])


_APX_NP, _APY_NP, _ST_NP = _anchor_consts()


def _sigmoid(x):
    return 1.0 / (1.0 + jnp.exp(-x))


def _softplus(x):
    return jnp.maximum(x, 0.0) + jnp.log1p(jnp.exp(-jnp.abs(x)))


def _bce(logits, targets):
    return (jnp.maximum(logits, 0.0) - logits * targets
            + jnp.log1p(jnp.exp(-jnp.abs(logits))))


def _atan_pos(x):
    big = x > 2.414213562373095
    mid = x > 0.4142135623730950
    xr = jnp.where(big, -1.0 / x, jnp.where(mid, (x - 1.0) / (x + 1.0), x))
    base = jnp.where(big, 0.5 * np.pi, jnp.where(mid, 0.25 * np.pi, 0.0))
    z = xr * xr
    p = 8.05374449538e-2
    p = p * z - 1.38776856032e-1
    p = p * z + 1.99777106478e-1
    p = p * z - 3.33329491539e-1
    return base + xr + xr * z * p


def _img_kernel(p3_ref, p4_ref, p5_ref, gtb_ref, gtt_ref, lab_ref,
                apx_ref, apy_ref, st_ref, out_ref, tk_ref, sel_ref):
    pred = jnp.concatenate([p3_ref[0], p4_ref[0], p5_ref[0]], axis=1)
    apx = apx_ref[...]
    apy = apy_ref[...]
    st = st_ref[...]

    obj = pred[4:5]
    cls = pred[5:5 + _NUM_CLASSES]

    cx = (2.0 * _sigmoid(pred[0:1]) - 1.0) * st + apx
    cy = (2.0 * _sigmoid(pred[1:2]) - 1.0) * st + apy
    pw = _softplus(pred[2:3]) * st
    ph = _softplus(pred[3:4]) * st
    px1 = cx - 0.5 * pw
    px2 = cx + 0.5 * pw
    py1 = cy - 0.5 * ph
    py2 = cy + 0.5 * ph

    gtb = gtb_ref[0]
    gcx = gtb[:, 0:1] * _IMG
    gcy = gtb[:, 1:2] * _IMG
    gww = gtb[:, 2:3] * _IMG
    ghh = gtb[:, 3:4] * _IMG
    gx1 = gcx - 0.5 * gww
    gx2 = gcx + 0.5 * gww
    gy1 = gcy - 0.5 * ghh
    gy2 = gcy + 0.5 * ghh
    gw = jnp.clip(gx2 - gx1, _EPS)
    gh = jnp.clip(gy2 - gy1, _EPS)

    inv_st2 = 1.0 / (st * st)
    area_cells = (gw * gh) * inv_st2
    gate = (area_cells >= _AREA_MIN) & (area_cells <= _AREA_MAX)
    dx = jnp.abs(apx - gcx)
    dy = jnp.abs(apy - gcy)
    r = _CENTER_RADIUS * st
    in_center = (dx < r) & (dy < r)
    in_box = (apx > gx1) & (apx < gx2) & (apy > gy1) & (apy < gy2)
    cand = gate & (in_center | in_box)

    a1 = jnp.clip(gx2 - gx1, 0.0) * jnp.clip(gy2 - gy1, 0.0)
    a2 = jnp.clip(px2 - px1, 0.0) * jnp.clip(py2 - py1, 0.0)
    ix1 = jnp.maximum(gx1, px1)
    iy1 = jnp.maximum(gy1, py1)
    ix2 = jnp.minimum(gx2, px2)
    iy2 = jnp.minimum(gy2, py2)
    inter = jnp.clip(ix2 - ix1, 0.0) * jnp.clip(iy2 - iy1, 0.0)
    iou = jnp.clip(inter / (a1 + a2 - inter + _EPS), 0.0, 1.0)

    pobj = _sigmoid(obj)
    t_exp = jnp.exp(-cls)
    sig_c = 1.0 / (1.0 + t_exp)
    omp = 1.0 - sig_c * pobj + _EPS
    y1 = omp[0:40] * omp[40:80]
    y2 = y1[0:20] * y1[20:40]
    y3 = y2[0:10] * y2[10:20]
    s_all = jnp.sum(jnp.log(y3), axis=0, keepdims=True)

    labs = lab_ref[0]
    c_iota = lax.broadcasted_iota(jnp.int32, (_NUM_CLASSES, _G), 0)
    oh_t = (labs == c_iota).astype(jnp.float32)
    dims = (((0,), (0,)), ((), ()))
    oh_b = oh_t.astype(jnp.bfloat16)
    c_hi = cls.astype(jnp.bfloat16)
    c_lo = (cls - c_hi.astype(jnp.float32)).astype(jnp.bfloat16)
    clsg = (lax.dot_general(oh_b, c_hi, dims,
                            preferred_element_type=jnp.float32)
            + lax.dot_general(oh_b, c_lo, dims,
                              preferred_element_type=jnp.float32))
    pg = _sigmoid(clsg) * pobj
    cls_sel = jnp.log(pg + _EPS) - jnp.log(1.0 - pg + _EPS)
    cls_cost = -(cls_sel + s_all)

    lpw = jnp.log(jnp.clip(px2 - px1, _EPS))
    lph = jnp.log(jnp.clip(py2 - py1, _EPS))
    lgw = jnp.log(gw)
    lgh = jnp.log(gh)
    size_cost = jnp.abs(lpw - lgw) + jnp.abs(lph - lgh)
    ar_cost = jnp.abs((lpw - lph) - (lgw - lgh))
    center_cost = jnp.sqrt(dx * dx + dy * dy) / st

    cost = (_IOU_COST_W * (1.0 - iou) + _ASSIGN_CLS_W * cls_cost
            + _SIZE_PRIOR_W * size_cost + _AR_PRIOR_W * ar_cost
            + _CENTER_COST_W * center_cost
            + jnp.where(cand, 0.0, 1e5))

    tk_ref[...] = -jnp.where(cand, iou, 0.0)
    sel_ref[...] = jnp.where(cand, cost, jnp.inf)
    cid20 = lax.broadcasted_iota(jnp.int32, (_G, _TOPK), 1)
    half = _A // 2

    def ext_step(rr, tmins, smins):
        tw = tk_ref[...]
        mnt = jnp.min(jnp.minimum(tw[:, :half], tw[:, half:]), axis=1,
                      keepdims=True)
        tk_ref[...] = jnp.where(tw == mnt, jnp.inf, tw)
        sw = sel_ref[...]
        mns = jnp.min(jnp.minimum(sw[:, :half], sw[:, half:]), axis=1,
                      keepdims=True)
        sel_ref[...] = jnp.where(sw == mns, jnp.inf, sw)
        return (jnp.where(cid20 == rr, mnt, tmins),
                jnp.where(cid20 == rr, mns, smins))

    def ext_body(it, carry):
        tmins, smins = ext_step(2 * it, *carry)
        return ext_step(2 * it + 1, tmins, smins)

    minf = jnp.full((_G, _TOPK), jnp.inf, jnp.float32)
    tmins, smins = lax.fori_loop(0, _TOPK // 2, ext_body, (minf, minf))
    tsum = jnp.sum(jnp.maximum(-tmins, 0.0), axis=1, keepdims=True)
    dyn_k = jnp.clip(tsum.astype(jnp.int32), 1, _TOPK)
    j20 = lax.broadcasted_iota(jnp.int32, (_G, _TOPK), 1)
    tau = jnp.sum(jnp.where(j20 == dyn_k - 1, smins, 0.0),
                  axis=1, keepdims=True)
    matched = cand & (cost <= tau)

    cost_m = jnp.where(matched, cost, jnp.inf)
    mn0 = jnp.min(cost_m, axis=0, keepdims=True)
    gmask = (cost_m == mn0).astype(jnp.float32)
    fg = jnp.max(matched.astype(jnp.float32), axis=0, keepdims=True)

    iou_at = jnp.sum(iou * gmask, axis=0, keepdims=True)
    xlab = jnp.sum(clsg * gmask, axis=0, keepdims=True)
    gtt = gtt_ref[0]
    gdims = (((1,), (0,)), ((), ()))
    g_hi = gtt.astype(jnp.bfloat16)
    g_lo = (gtt - g_hi.astype(jnp.float32)).astype(jnp.bfloat16)
    gm_b = gmask.astype(jnp.bfloat16)
    tcoord = (lax.dot_general(g_hi, gm_b, gdims,
                              preferred_element_type=jnp.float32)
              + lax.dot_general(g_lo, gm_b, gdims,
                                preferred_element_type=jnp.float32))
    tx1 = tcoord[0:1]
    tx2 = tcoord[1:2]
    ty1 = tcoord[2:3]
    ty2 = tcoord[3:4]

    nfg = jnp.maximum(jnp.sum(fg), 1.0)

    pwc = jnp.clip(px2 - px1, _EPS)
    phc = jnp.clip(py2 - py1, _EPS)
    twc = jnp.clip(tx2 - tx1, _EPS)
    thc = jnp.clip(ty2 - ty1, _EPS)
    iw = jnp.clip(jnp.minimum(px2, tx2) - jnp.maximum(px1, tx1), 0.0)
    ih = jnp.clip(jnp.minimum(py2, ty2) - jnp.maximum(py1, ty1), 0.0)
    inter2 = iw * ih
    union2 = pwc * phc + twc * thc - inter2 + _EPS
    iou2 = inter2 / union2
    cd = (((px1 + px2) * 0.5 - (tx1 + tx2) * 0.5) ** 2
          + ((py1 + py2) * 0.5 - (ty1 + ty2) * 0.5) ** 2)
    cw = jnp.maximum(px2, tx2) - jnp.minimum(px1, tx1)
    chh = jnp.maximum(py2, ty2) - jnp.minimum(py1, ty1)
    c2 = cw * cw + chh * chh + _EPS
    v = 4.0 / _PI2 * (_atan_pos(twc / thc) - _atan_pos(pwc / phc)) ** 2
    alpha = v / (v - iou2 + 1.0 + _EPS)
    ciou = iou2 - cd / c2 - alpha * v
    loss_box = jnp.sum(fg * (1.0 - ciou)) / nfg

    obj_t = fg * iou_at
    loss_obj = jnp.sum(_bce(obj, obj_t)) / float(_A)
    off = _CLS_SMOOTH / (_NUM_CLASSES - 1)
    scale = 1.0 - _CLS_SMOOTH - off
    e_abs = jnp.exp(-jnp.abs(cls))
    op1 = 1.0 + e_abs
    w1 = op1[0:40] * op1[40:80]
    w2 = w1[0:20] * w1[20:40]
    w3 = w2[0:10] * w2[10:20]
    sp_sum = (jnp.sum(jnp.maximum(cls, 0.0), axis=0, keepdims=True)
              + jnp.sum(jnp.log(w3), axis=0, keepdims=True))
    sumx = jnp.sum(cls, axis=0, keepdims=True)
    bce_mean = (sp_sum - off * sumx - scale * xlab) / float(_NUM_CLASSES)
    loss_cls = jnp.sum(fg * bce_mean) / nfg

    loss = (_LAMBDA_BOX * loss_box + _LAMBDA_OBJ * loss_obj
            + _LAMBDA_CLS * loss_cls)
    out_ref[...] = loss[None, None, None]


def kernel(p3, p4, p5, gt_boxes, gt_labels):
    b = p3.shape[0]
    p3f = p3.reshape(b, 5 + _NUM_CLASSES, _SIZES[0][0] * _SIZES[0][1])
    p4f = p4.reshape(b, 5 + _NUM_CLASSES, _SIZES[1][0] * _SIZES[1][1])
    p5f = p5.reshape(b, 5 + _NUM_CLASSES, _SIZES[2][0] * _SIZES[2][1])
    labs = gt_labels.astype(jnp.int32).reshape(b, 1, _G)
    cxg = gt_boxes[..., 0] * _IMG
    cyg = gt_boxes[..., 1] * _IMG
    wwg = gt_boxes[..., 2] * _IMG
    hhg = gt_boxes[..., 3] * _IMG
    gtt = jnp.stack([cxg - 0.5 * wwg, cxg + 0.5 * wwg,
                     cyg - 0.5 * hhg, cyg + 0.5 * hhg], axis=1)
    apx = jnp.asarray(_APX_NP)
    apy = jnp.asarray(_APY_NP)
    st = jnp.asarray(_ST_NP)
    ch = 5 + _NUM_CLASSES
    out = pl.pallas_call(
        _img_kernel,
        grid=(b,),
        in_specs=[
            pl.BlockSpec((1, ch, p3f.shape[2]), lambda i: (i, 0, 0)),
            pl.BlockSpec((1, ch, p4f.shape[2]), lambda i: (i, 0, 0)),
            pl.BlockSpec((1, ch, p5f.shape[2]), lambda i: (i, 0, 0)),
            pl.BlockSpec((1, _G, 4), lambda i: (i, 0, 0)),
            pl.BlockSpec((1, 4, _G), lambda i: (i, 0, 0)),
            pl.BlockSpec((1, 1, _G), lambda i: (i, 0, 0)),
            pl.BlockSpec((1, _A), lambda i: (0, 0)),
            pl.BlockSpec((1, _A), lambda i: (0, 0)),
            pl.BlockSpec((1, _A), lambda i: (0, 0)),
        ],
        out_specs=pl.BlockSpec((1, 1, 1), lambda i: (i, 0, 0)),
        out_shape=jax.ShapeDtypeStruct((b, 1, 1), jnp.float32),
        scratch_shapes=[pltpu.VMEM((_G, _A), jnp.float32),
                        pltpu.VMEM((_G, _A), jnp.float32)],
    )(p3f, p4f, p5f, gt_boxes, gtt, labs, apx, apy, st)
    return jnp.mean(out)
